# Initial kernel scaffold; baseline (speedup 1.0000x reference)
import jax
import jax.numpy as jnp
from jax.experimental import pallas as pl


def kernel(input, target, pre_input, it, bi, ti):
    raise NotImplementedError("write your pallas kernel here")


# speedup vs baseline: 2.1217x; 2.1217x over previous
import jax
import jax.numpy as jnp
from jax.experimental import pallas as pl

_START_WARM = 1000
_END_WARM = 5000
_TOP_P = 0.15


def _raw_stage(x_ref, t_ref, px_ref, out_ref):
    x = x_ref[...]
    px = px_ref[...]
    tgt = t_ref[...]

    cidx = jax.lax.broadcasted_iota(jnp.int32, x.shape, 0)
    isrest = cidx > 0

    m = jnp.max(x, axis=0, keepdims=True)
    e = jnp.exp(x - m)
    z = jnp.sum(e, axis=0, keepdims=True)
    p = jnp.clip(e / z, 1e-7, 1.0)
    p0 = p[0:1, :]
    srest = jnp.sum(jnp.where(isrest, p, 0.0), axis=0, keepdims=True)

    mrest = jnp.max(jnp.where(isrest, px, -jnp.inf), axis=0, keepdims=True)
    uni = jnp.logical_or(tgt != 0, mrest > px[0:1, :])

    mlogc = -jnp.log(jnp.float32(1e-4))
    out_ref[...] = jnp.where(uni, p0, srest) * mlogc


def _select_stage(r_ref, loss_ref, mean_ref, *, k, n):
    r = r_ref[...]
    bits = jax.lax.bitcast_convert_type(r, jnp.int32)
    total = jnp.sum(r)

    def body(_, lohi):
        lo, hi = lohi
        mid = lo + (hi - lo) // 2
        cnt = jnp.sum((bits >= mid).astype(jnp.int32))
        take = cnt >= k
        return (jnp.where(take, mid, lo), jnp.where(take, hi, mid))

    lo, _ = jax.lax.fori_loop(
        0, 31, body, (jnp.int32(0), jnp.int32(0x7F800000)))
    t = jax.lax.bitcast_convert_type(lo, jnp.float32)

    gt = r > t
    cnt_gt = jnp.sum(gt.astype(jnp.int32))
    sum_gt = jnp.sum(jnp.where(gt, r, 0.0))
    sum_top = sum_gt + (jnp.float32(k) - cnt_gt.astype(jnp.float32)) * t
    loss_ref[...] = jnp.full(loss_ref.shape, sum_top / jnp.float32(k),
                             jnp.float32)
    mean_ref[...] = jnp.full(mean_ref.shape, total / jnp.float32(n),
                             jnp.float32)


def kernel(input, target, pre_input, it, bi, ti):
    c = input.shape[1]
    n = input.shape[2] * input.shape[3]
    k = int(n * _TOP_P)
    blk = 9216
    nb = n // blk

    x2 = input.reshape(c, n)
    px2 = pre_input.reshape(c, n)
    t2 = target.reshape(1, n)

    raw = pl.pallas_call(
        _raw_stage,
        grid=(nb,),
        in_specs=[
            pl.BlockSpec((c, blk), lambda i: (0, i)),
            pl.BlockSpec((1, blk), lambda i: (0, i)),
            pl.BlockSpec((c, blk), lambda i: (0, i)),
        ],
        out_specs=pl.BlockSpec((1, blk), lambda i: (0, i)),
        out_shape=jax.ShapeDtypeStruct((1, n), jnp.float32),
    )(x2, t2, px2)

    import functools
    sel = functools.partial(_select_stage, k=k, n=n)
    loss2, mean2 = pl.pallas_call(
        sel,
        out_shape=[
            jax.ShapeDtypeStruct((1, 128), jnp.float32),
            jax.ShapeDtypeStruct((1, 128), jnp.float32),
        ],
    )(raw.reshape(n // 128, 128))

    mean_top = loss2[0, 0]
    mean_all = mean2[0, 0]

    warm = it < _START_WARM
    this_p = jnp.where(
        it > _END_WARM,
        _TOP_P,
        _TOP_P + (1 - _TOP_P) * ((_END_WARM - it) / (_END_WARM - _START_WARM)),
    )
    loss_out = jnp.where(warm, mean_all, mean_top)
    p_out = jnp.where(warm, jnp.asarray(1.0, dtype=jnp.float32),
                      this_p.astype(jnp.float32))
    return (loss_out, jnp.asarray(p_out, dtype=jnp.float32))


# speedup vs baseline: 6.2409x; 2.9415x over previous
import jax
import jax.numpy as jnp
from jax.experimental import pallas as pl

_START_WARM = 1000
_END_WARM = 5000
_TOP_P = 0.15


def _raw_stage(x_ref, t_ref, px_ref, out_ref):
    x = x_ref[...]
    px = px_ref[...]
    tgt = t_ref[...]

    cidx = jax.lax.broadcasted_iota(jnp.int32, x.shape, 0)
    isrest = cidx > 0

    m = jnp.max(x, axis=0, keepdims=True)
    e = jnp.exp(x - m)
    z = jnp.sum(e, axis=0)
    p = jnp.clip(e * (1.0 / z)[None], 1e-7, 1.0)
    p0 = p[0]
    srest = jnp.sum(jnp.where(isrest, p, 0.0), axis=0)

    mrest = jnp.max(jnp.where(isrest, px, -jnp.inf), axis=0)
    uni = jnp.logical_or(tgt != 0, mrest > px[0])

    mlogc = -jnp.log(jnp.float32(1e-4))
    out_ref[...] = jnp.where(uni, p0, srest) * mlogc


def _select_stage(r_ref, loss_ref, mean_ref, *, k, n):
    r = r_ref[...]
    bits = jax.lax.bitcast_convert_type(r, jnp.int32)
    total = jnp.sum(r)

    def body(_, lohi):
        lo, hi = lohi
        mid = lo + (hi - lo) // 2
        cnt = jnp.sum((bits >= mid).astype(jnp.int32))
        take = cnt >= k
        return (jnp.where(take, mid, lo), jnp.where(take, hi, mid))

    lo, _ = jax.lax.fori_loop(
        0, 31, body, (jnp.int32(0), jnp.int32(0x7F800000)))
    t = jax.lax.bitcast_convert_type(lo, jnp.float32)

    gt = r > t
    cnt_gt = jnp.sum(gt.astype(jnp.int32))
    sum_gt = jnp.sum(jnp.where(gt, r, 0.0))
    sum_top = sum_gt + (jnp.float32(k) - cnt_gt.astype(jnp.float32)) * t
    loss_ref[...] = jnp.full(loss_ref.shape, sum_top / jnp.float32(k),
                             jnp.float32)
    mean_ref[...] = jnp.full(mean_ref.shape, total / jnp.float32(n),
                             jnp.float32)


def kernel(input, target, pre_input, it, bi, ti):
    c = input.shape[1]
    h, w = input.shape[2], input.shape[3]
    n = h * w
    k = int(n * _TOP_P)
    hb = 32
    nb = h // hb

    x3 = input[0]
    px3 = pre_input[0]
    t2 = target[0]

    raw = pl.pallas_call(
        _raw_stage,
        grid=(nb,),
        in_specs=[
            pl.BlockSpec((c, hb, w), lambda i: (0, i, 0)),
            pl.BlockSpec((hb, w), lambda i: (i, 0)),
            pl.BlockSpec((c, hb, w), lambda i: (0, i, 0)),
        ],
        out_specs=pl.BlockSpec((hb, w), lambda i: (i, 0)),
        out_shape=jax.ShapeDtypeStruct((h, w), jnp.float32),
    )(x3, t2, px3)

    import functools
    sel = functools.partial(_select_stage, k=k, n=n)
    loss2, mean2 = pl.pallas_call(
        sel,
        out_shape=[
            jax.ShapeDtypeStruct((1, 128), jnp.float32),
            jax.ShapeDtypeStruct((1, 128), jnp.float32),
        ],
    )(raw)

    mean_top = loss2[0, 0]
    mean_all = mean2[0, 0]

    warm = it < _START_WARM
    this_p = jnp.where(
        it > _END_WARM,
        _TOP_P,
        _TOP_P + (1 - _TOP_P) * ((_END_WARM - it) / (_END_WARM - _START_WARM)),
    )
    loss_out = jnp.where(warm, mean_all, mean_top)
    p_out = jnp.where(warm, jnp.asarray(1.0, dtype=jnp.float32),
                      this_p.astype(jnp.float32))
    return (loss_out, jnp.asarray(p_out, dtype=jnp.float32))


# speedup vs baseline: 6.7181x; 1.0765x over previous
import functools

import jax
import jax.numpy as jnp
from jax.experimental import pallas as pl
from jax.experimental.pallas import tpu as pltpu

_START_WARM = 1000
_END_WARM = 5000
_TOP_P = 0.15


def _fused(x_ref, t_ref, px_ref, loss_ref, mean_ref, raw_ref, *, k, n, nb, hb):
    i = pl.program_id(0)
    x = x_ref[...]
    px = px_ref[...]
    tgt = t_ref[...]

    m = jnp.max(x, axis=0, keepdims=True)
    e = jnp.exp(x - m)
    z = jnp.sum(e, axis=0)
    p = jnp.clip(e * (1.0 / z)[None], 1e-7, 1.0)
    p0 = p[0]
    srest = jnp.sum(p, axis=0) - p0

    mall = jnp.max(px, axis=0)
    uni = jnp.logical_or(tgt != 0, mall > px[0])

    mlogc = -jnp.log(jnp.float32(1e-4))
    raw_ref[pl.ds(i * hb, hb), :] = jnp.where(uni, p0, srest) * mlogc

    @pl.when(i == nb - 1)
    def _select():
        r = raw_ref[...]
        bits = jax.lax.bitcast_convert_type(r, jnp.int32)
        total = jnp.sum(r)

        def body(_, lohi):
            lo, hi = lohi
            mid = lo + (hi - lo) // 2
            cnt = jnp.sum((bits >= mid).astype(jnp.int32))
            take = cnt >= k
            return (jnp.where(take, mid, lo), jnp.where(take, hi, mid))

        lo, _ = jax.lax.fori_loop(
            0, 31, body, (jnp.int32(0), jnp.int32(0x7F800000)))
        t = jax.lax.bitcast_convert_type(lo, jnp.float32)

        gt = r > t
        cnt_gt = jnp.sum(gt.astype(jnp.int32))
        sum_gt = jnp.sum(jnp.where(gt, r, 0.0))
        sum_top = sum_gt + (jnp.float32(k) - cnt_gt.astype(jnp.float32)) * t
        loss_ref[...] = jnp.full(loss_ref.shape, sum_top / jnp.float32(k),
                                 jnp.float32)
        mean_ref[...] = jnp.full(mean_ref.shape, total / jnp.float32(n),
                                 jnp.float32)


def kernel(input, target, pre_input, it, bi, ti):
    c = input.shape[1]
    h, w = input.shape[2], input.shape[3]
    n = h * w
    k = int(n * _TOP_P)
    hb = 48
    nb = h // hb

    x3 = input[0]
    px3 = pre_input[0]
    t2 = target[0]

    body = functools.partial(_fused, k=k, n=n, nb=nb, hb=hb)
    loss2, mean2 = pl.pallas_call(
        body,
        grid=(nb,),
        in_specs=[
            pl.BlockSpec((c, hb, w), lambda i: (0, i, 0)),
            pl.BlockSpec((hb, w), lambda i: (i, 0)),
            pl.BlockSpec((c, hb, w), lambda i: (0, i, 0)),
        ],
        out_specs=[
            pl.BlockSpec((1, 128), lambda i: (0, 0)),
            pl.BlockSpec((1, 128), lambda i: (0, 0)),
        ],
        out_shape=[
            jax.ShapeDtypeStruct((1, 128), jnp.float32),
            jax.ShapeDtypeStruct((1, 128), jnp.float32),
        ],
        scratch_shapes=[pltpu.VMEM((h, w), jnp.float32)],
    )(x3, t2, px3)

    mean_top = loss2[0, 0]
    mean_all = mean2[0, 0]

    warm = it < _START_WARM
    this_p = jnp.where(
        it > _END_WARM,
        _TOP_P,
        _TOP_P + (1 - _TOP_P) * ((_END_WARM - it) / (_END_WARM - _START_WARM)),
    )
    loss_out = jnp.where(warm, mean_all, mean_top)
    p_out = jnp.where(warm, jnp.asarray(1.0, dtype=jnp.float32),
                      this_p.astype(jnp.float32))
    return (loss_out, jnp.asarray(p_out, dtype=jnp.float32))


# speedup vs baseline: 7.1379x; 1.0625x over previous
import functools

import jax
import jax.numpy as jnp
from jax.experimental import pallas as pl
from jax.experimental.pallas import tpu as pltpu

_START_WARM = 1000
_END_WARM = 5000
_TOP_P = 0.15


def _fused(x_ref, t_ref, px_ref, loss_ref, mean_ref, raw_ref, *, k, n, nb, hb):
    i = pl.program_id(0)
    x = x_ref[...]
    px = px_ref[...]
    tgt = t_ref[...]

    m = jnp.max(x, axis=0, keepdims=True)
    e = jnp.exp(x - m)
    z = jnp.sum(e, axis=0)
    p = jnp.clip(e * (1.0 / z)[None], 1e-7, 1.0)
    p0 = p[0]
    srest = jnp.sum(p, axis=0) - p0

    mall = jnp.max(px, axis=0)
    uni = jnp.logical_or(tgt != 0, mall > px[0])

    mlogc = -jnp.log(jnp.float32(1e-4))
    raw_ref[pl.ds(i * hb, hb), :] = jnp.where(uni, p0, srest) * mlogc

    @pl.when(i == nb - 1)
    def _select():
        r = raw_ref[...]
        bits = jax.lax.bitcast_convert_type(r, jnp.int32)
        total = jnp.sum(r)

        lo0 = jnp.int32(0x35719787)
        hi0 = jnp.int32(0x41200000)

        def cond(st):
            lo, hi, cnt = st
            return jnp.logical_and(hi - lo > 512, cnt != k)

        def body(st):
            lo, hi, _ = st
            mid = lo + (hi - lo) // 2
            cnt = jnp.sum((bits >= mid).astype(jnp.int32))
            take = cnt >= k
            return (jnp.where(take, mid, lo), jnp.where(take, hi, mid), cnt)

        lo, _, _ = jax.lax.while_loop(cond, body, (lo0, hi0, jnp.int32(0)))
        t = jax.lax.bitcast_convert_type(lo, jnp.float32)

        gt = r > t
        cnt_gt = jnp.sum(gt.astype(jnp.int32))
        sum_gt = jnp.sum(jnp.where(gt, r, 0.0))
        sum_top = sum_gt + (jnp.float32(k) - cnt_gt.astype(jnp.float32)) * t
        loss_ref[...] = jnp.full(loss_ref.shape, sum_top / jnp.float32(k),
                                 jnp.float32)
        mean_ref[...] = jnp.full(mean_ref.shape, total / jnp.float32(n),
                                 jnp.float32)


def kernel(input, target, pre_input, it, bi, ti):
    c = input.shape[1]
    h, w = input.shape[2], input.shape[3]
    n = h * w
    k = int(n * _TOP_P)
    hb = 48
    nb = h // hb

    x3 = input[0]
    px3 = pre_input[0]
    t2 = target[0]

    body = functools.partial(_fused, k=k, n=n, nb=nb, hb=hb)
    loss2, mean2 = pl.pallas_call(
        body,
        grid=(nb,),
        in_specs=[
            pl.BlockSpec((c, hb, w), lambda i: (0, i, 0)),
            pl.BlockSpec((hb, w), lambda i: (i, 0)),
            pl.BlockSpec((c, hb, w), lambda i: (0, i, 0)),
        ],
        out_specs=[
            pl.BlockSpec((1, 128), lambda i: (0, 0)),
            pl.BlockSpec((1, 128), lambda i: (0, 0)),
        ],
        out_shape=[
            jax.ShapeDtypeStruct((1, 128), jnp.float32),
            jax.ShapeDtypeStruct((1, 128), jnp.float32),
        ],
        scratch_shapes=[pltpu.VMEM((h, w), jnp.float32)],
    )(x3, t2, px3)

    mean_top = loss2[0, 0]
    mean_all = mean2[0, 0]

    warm = it < _START_WARM
    this_p = jnp.where(
        it > _END_WARM,
        _TOP_P,
        _TOP_P + (1 - _TOP_P) * ((_END_WARM - it) / (_END_WARM - _START_WARM)),
    )
    loss_out = jnp.where(warm, mean_all, mean_top)
    p_out = jnp.where(warm, jnp.asarray(1.0, dtype=jnp.float32),
                      this_p.astype(jnp.float32))
    return (loss_out, jnp.asarray(p_out, dtype=jnp.float32))
